# baseline (speedup 1.0000x reference)
import functools

import jax
import jax.numpy as jnp
from jax import lax
from jax.experimental import pallas as pl
from jax.experimental.pallas import tpu as pltpu
from jax.experimental.pallas import tpu_sc as plsc

_T = 512
_SC_NC = 2
_SC_NS = 16
_SC_CH = 64


def _knn_kernel(n2, p2_ref, p1_ref, q2_ref,
                i0_ref, i1_ref, i2_ref, w0_ref, w1_ref, w2_ref):
    b = pl.program_id(0)
    t = p1_ref.shape[2]

    p1 = p1_ref[0]
    q1 = jnp.sum(p1 * p1, axis=0, keepdims=True)
    cc = lax.dot_general(p2_ref[0].astype(jnp.bfloat16),
                         p1.astype(jnp.bfloat16),
                         (((0,), (0,)), ((), ())),
                         preferred_element_type=jnp.float32)
    d = jnp.maximum((q1 + q2_ref[0]) - 2.0 * cc, 0.0)

    sub = lax.broadcasted_iota(jnp.int32, (n2, t), 0)
    inf = jnp.float32(jnp.inf)
    dcur = d
    ams, ws = [], []
    wsum = jnp.zeros((1, t), jnp.float32)
    for k in range(3):
        m = jnp.min(dcur, axis=0, keepdims=True)
        am = jnp.min(jnp.where(dcur == m, sub, n2), axis=0,
                     keepdims=True)
        wk = 1.0 / (m + 1e-5)
        wsum = wsum + wk
        ams.append(am)
        ws.append(wk)
        if k < 2:
            dcur = jnp.where(sub == am, inf, dcur)

    off = b * n2
    i0_ref[0] = ams[0] + off
    i1_ref[0] = ams[1] + off
    i2_ref[0] = ams[2] + off
    w0_ref[0] = ws[0] / wsum
    w1_ref[0] = ws[1] / wsum
    w2_ref[0] = ws[2] / wsum


def _sc_gather_kernel(nch, table_ref, i0_ref, i1_ref, i2_ref,
                      w0_ref, w1_ref, w2_ref, out_ref,
                      i0a, i1a, i2a, r0a, r1a, r2a, sema,
                      i0b, i1b, i2b, r0b, r1b, r2b, semb,
                      w0v, w1v, w2v, ov):
    ch = _SC_CH
    wid = lax.axis_index("s") * _SC_NC + lax.axis_index("c")
    base = wid * (nch * ch)
    bufs = ((i0a, i1a, i2a, r0a, r1a, r2a, sema),
            (i0b, i1b, i2b, r0b, r1b, r2b, semb))

    def fire(c, slot):
        i0v, i1v, i2v, r0v, r1v, r2v, sem = slot
        off = base + c * ch
        pltpu.sync_copy(i0_ref.at[pl.ds(off, ch)], i0v)
        pltpu.sync_copy(i1_ref.at[pl.ds(off, ch)], i1v)
        pltpu.sync_copy(i2_ref.at[pl.ds(off, ch)], i2v)
        return (pltpu.async_copy(table_ref.at[i0v], r0v, sem),
                pltpu.async_copy(table_ref.at[i1v], r1v, sem),
                pltpu.async_copy(table_ref.at[i2v], r2v, sem))

    def compute(c, slot):
        _, _, _, r0v, r1v, r2v, _ = slot
        off = base + c * ch
        pltpu.sync_copy(w0_ref.at[pl.ds(off, ch)], w0v)
        pltpu.sync_copy(w1_ref.at[pl.ds(off, ch)], w1v)
        pltpu.sync_copy(w2_ref.at[pl.ds(off, ch)], w2v)

        def group_body(jg, carry2):
            w0c = w0v[pl.ds(jg * 16, 16)]
            w1c = w1v[pl.ds(jg * 16, 16)]
            w2c = w2v[pl.ds(jg * 16, 16)]

            def out_body(j16, carry3):
                jl = lax.broadcast(j16, (16,))
                w0s = w0c.at[jl].get(mode="promise_in_bounds")
                w1s = w1c.at[jl].get(mode="promise_in_bounds")
                w2s = w2c.at[jl].get(mode="promise_in_bounds")
                j = jg * 16 + j16
                for cc16 in range(16):
                    sl = pl.ds(cc16 * 16, 16)
                    ov[j, sl] = (r0v[j, sl] * w0s + r1v[j, sl] * w1s
                                 + r2v[j, sl] * w2s)
                return carry3

            lax.fori_loop(0, 16, out_body, 0)
            return carry2

        lax.fori_loop(0, ch // 16, group_body, 0)
        pltpu.sync_copy(ov, out_ref.at[pl.ds(off, ch)])

    handles = {0: fire(0, bufs[0])}
    for c in range(nch):
        if c + 1 < nch:
            handles[(c + 1) % 2] = fire(c + 1, bufs[(c + 1) % 2])
        for h in handles[c % 2]:
            h.wait()
        compute(c, bufs[c % 2])


def _mm1_kernel(nb, ni, f1_ref, g_ref, w1a_ref, w1b_ref,
                y1_ref, ssum_ref, ssq_ref, acc, accsq):
    b = pl.program_id(0)
    i = pl.program_id(1)

    @pl.when((b == 0) & (i == 0))
    def _init():
        acc[...] = jnp.zeros_like(acc)
        accsq[...] = jnp.zeros_like(accsq)

    g = g_ref[0].astype(jnp.bfloat16)
    y = (jnp.dot(w1a_ref[...], f1_ref[0].astype(jnp.bfloat16),
                 preferred_element_type=jnp.float32)
         + lax.dot_general(w1b_ref[...], g, (((1,), (1,)), ((), ())),
                           preferred_element_type=jnp.float32))
    y1_ref[0] = y
    acc[...] += y
    accsq[...] += y * y

    @pl.when((b == nb - 1) & (i == ni - 1))
    def _fin():
        ssum_ref[...] = jnp.broadcast_to(
            jnp.sum(acc[...], axis=1, keepdims=True), ssum_ref.shape)
        ssq_ref[...] = jnp.broadcast_to(
            jnp.sum(accsq[...], axis=1, keepdims=True), ssq_ref.shape)


def _bn_mm2_kernel(nb, ni, y1_ref, w2_ref, sc_ref, sh_ref,
                   y2_ref, ssum_ref, ssq_ref, acc, accsq):
    b = pl.program_id(0)
    i = pl.program_id(1)

    @pl.when((b == 0) & (i == 0))
    def _init():
        acc[...] = jnp.zeros_like(acc)
        accsq[...] = jnp.zeros_like(accsq)

    z = jnp.maximum(y1_ref[0] * sc_ref[:, 0:1] + sh_ref[:, 0:1], 0.0)
    y = jnp.dot(w2_ref[...], z.astype(jnp.bfloat16),
                preferred_element_type=jnp.float32)
    y2_ref[0] = y
    acc[...] += y
    accsq[...] += y * y

    @pl.when((b == nb - 1) & (i == ni - 1))
    def _fin():
        ssum_ref[...] = jnp.broadcast_to(
            jnp.sum(acc[...], axis=1, keepdims=True), ssum_ref.shape)
        ssq_ref[...] = jnp.broadcast_to(
            jnp.sum(accsq[...], axis=1, keepdims=True), ssq_ref.shape)


def _bn_out_kernel(y2_ref, sc_ref, sh_ref, o_ref):
    o_ref[0] = jnp.maximum(y2_ref[0] * sc_ref[:, 0:1] + sh_ref[:, 0:1], 0.0)


def _bn_out_kernel2(y2_ref, sc_ref, sh_ref, prev_ref, o_ref):
    o_ref[0] = jnp.maximum(y2_ref[0] * sc_ref[:, 0:1] + sh_ref[:, 0:1], 0.0)


def _knn_call(p2, p1, q2col, N2, T):
    Bh, _, N1 = p1.shape
    i32 = jnp.int32
    f32 = jnp.float32
    grid = (Bh, N1 // T)
    return pl.pallas_call(
        functools.partial(_knn_kernel, N2),
        grid=grid,
        in_specs=[
            pl.BlockSpec((1, 3, N2), lambda b, i: (b, 0, 0)),
            pl.BlockSpec((1, 3, T), lambda b, i: (b, 0, i)),
            pl.BlockSpec((1, N2, 1), lambda b, i: (b, 0, 0)),
        ],
        out_specs=[pl.BlockSpec((1, 1, T), lambda b, i: (b, 0, i))] * 6,
        out_shape=[jax.ShapeDtypeStruct((Bh, 1, N1), i32)] * 3
        + [jax.ShapeDtypeStruct((Bh, 1, N1), f32)] * 3,
    )(p2, p1, q2col)


def _gather_call(table, knn6, C2):
    Bh, _, N1 = knn6[0].shape
    rows = Bh * N1
    nw = _SC_NC * _SC_NS
    nch = rows // (nw * _SC_CH)
    i32 = jnp.int32
    f32 = jnp.float32
    i0, i1, i2, w0, w1, w2 = [a.reshape(rows) for a in knn6]
    gath = pl.kernel(
        functools.partial(_sc_gather_kernel, nch),
        out_type=jax.ShapeDtypeStruct((rows, C2), f32),
        mesh=plsc.VectorSubcoreMesh(core_axis_name="c", subcore_axis_name="s",
                                    num_cores=_SC_NC, num_subcores=_SC_NS),
        scratch_types=(
            [pltpu.VMEM((_SC_CH,), i32)] * 3
            + [pltpu.VMEM((_SC_CH, C2), f32)] * 3
            + [pltpu.SemaphoreType.DMA]
            + [pltpu.VMEM((_SC_CH,), i32)] * 3
            + [pltpu.VMEM((_SC_CH, C2), f32)] * 3
            + [pltpu.SemaphoreType.DMA]
            + [pltpu.VMEM((_SC_CH,), f32)] * 3
            + [pltpu.VMEM((_SC_CH, C2), f32)]
        ),
    )(table, i0, i1, i2, w0, w1, w2)
    return gath.reshape(Bh, N1, C2)


def _mm1_call(f1, f2i, W1a, W1b, T):
    Bh, C1, N1 = f1.shape
    C2 = f2i.shape[2]
    D1 = W1a.shape[0]
    NI = N1 // T
    f32 = jnp.float32
    grid = (Bh, NI)
    return pl.pallas_call(
        functools.partial(_mm1_kernel, Bh, NI),
        grid=grid,
        in_specs=[
            pl.BlockSpec((1, C1, T), lambda b, i: (b, 0, i)),
            pl.BlockSpec((1, T, C2), lambda b, i: (b, i, 0)),
            pl.BlockSpec((D1, C1), lambda b, i: (0, 0)),
            pl.BlockSpec((D1, C2), lambda b, i: (0, 0)),
        ],
        out_specs=[
            pl.BlockSpec((1, D1, T), lambda b, i: (b, 0, i)),
            pl.BlockSpec((D1, 128), lambda b, i: (0, 0)),
            pl.BlockSpec((D1, 128), lambda b, i: (0, 0)),
        ],
        out_shape=[
            jax.ShapeDtypeStruct((Bh, D1, N1), f32),
            jax.ShapeDtypeStruct((D1, 128), f32),
            jax.ShapeDtypeStruct((D1, 128), f32),
        ],
        scratch_shapes=[
            pltpu.VMEM((D1, T), f32),
            pltpu.VMEM((D1, T), f32),
        ],
    )(f1, f2i, W1a, W1b)


def _mm2_call(y1, W2bf, sc1, sh1, T):
    Bh, D1, N1 = y1.shape
    D2 = W2bf.shape[0]
    NI = N1 // T
    f32 = jnp.float32
    grid = (Bh, NI)
    return pl.pallas_call(
        functools.partial(_bn_mm2_kernel, Bh, NI),
        grid=grid,
        in_specs=[
            pl.BlockSpec((1, D1, T), lambda b, i: (b, 0, i)),
            pl.BlockSpec((D2, D1), lambda b, i: (0, 0)),
            pl.BlockSpec((D1, 128), lambda b, i: (0, 0)),
            pl.BlockSpec((D1, 128), lambda b, i: (0, 0)),
        ],
        out_specs=[
            pl.BlockSpec((1, D2, T), lambda b, i: (b, 0, i)),
            pl.BlockSpec((D2, 128), lambda b, i: (0, 0)),
            pl.BlockSpec((D2, 128), lambda b, i: (0, 0)),
        ],
        out_shape=[
            jax.ShapeDtypeStruct((Bh, D2, N1), f32),
            jax.ShapeDtypeStruct((D2, 128), f32),
            jax.ShapeDtypeStruct((D2, 128), f32),
        ],
        scratch_shapes=[
            pltpu.VMEM((D2, T), f32),
            pltpu.VMEM((D2, T), f32),
        ],
    )(y1, W2bf, sc1, sh1)


def _out_call(y2a, y2b, sc2, sh2, T):
    H, D2, N1 = y2a.shape
    B = 2 * H
    grid = (H, N1 // T)
    half = pl.pallas_call(
        _bn_out_kernel,
        grid=grid,
        in_specs=[
            pl.BlockSpec((1, D2, T), lambda b, i: (b, 0, i)),
            pl.BlockSpec((D2, 128), lambda b, i: (0, 0)),
            pl.BlockSpec((D2, 128), lambda b, i: (0, 0)),
        ],
        out_specs=pl.BlockSpec((1, D2, T), lambda b, i: (b, 0, i)),
        out_shape=jax.ShapeDtypeStruct((B, D2, N1), jnp.float32),
    )(y2a, sc2, sh2)
    return pl.pallas_call(
        _bn_out_kernel2,
        grid=grid,
        in_specs=[
            pl.BlockSpec((1, D2, T), lambda b, i: (b, 0, i)),
            pl.BlockSpec((D2, 128), lambda b, i: (0, 0)),
            pl.BlockSpec((D2, 128), lambda b, i: (0, 0)),
            pl.BlockSpec(memory_space=pltpu.MemorySpace.HBM),
        ],
        out_specs=pl.BlockSpec((1, D2, T), lambda b, i: (b + H, 0, i)),
        out_shape=jax.ShapeDtypeStruct((B, D2, N1), jnp.float32),
        input_output_aliases={3: 0},
    )(y2b, sc2, sh2, half)


def kernel(points1, features1, points2, features2,
           W1, b1, g1, be1, W2, b2, g2, be2):
    B, _, N1 = points1.shape
    N2 = points2.shape[2]
    C1 = features1.shape[1]
    C2 = features2.shape[1]
    D1 = W1.shape[0]
    D2 = W2.shape[0]
    T = _T
    f32 = jnp.float32
    bf16 = jnp.bfloat16

    sq2 = jnp.sum(points2 * points2, axis=1, keepdims=True)
    q2col = jnp.transpose(sq2, (0, 2, 1))

    W1a = W1[:, :C1].astype(bf16)
    W1b = W1[:, C1:].astype(bf16)
    W2bf = W2.astype(bf16)

    H = B // 2
    tables = [jnp.transpose(features2[h * H:(h + 1) * H], (0, 2, 1))
              .reshape(H * N2, C2) for h in range(2)]

    knn = [_knn_call(points2[h * H:(h + 1) * H], points1[h * H:(h + 1) * H],
                     q2col[h * H:(h + 1) * H], N2, T) for h in range(2)]
    f2i = [_gather_call(tables[h], knn[h], C2) for h in range(2)]
    m1 = [_mm1_call(features1[h * H:(h + 1) * H], f2i[h], W1a, W1b, T)
          for h in range(2)]

    cnt = f32(B * N1)
    ssum1 = m1[0][1][:, 0] + m1[1][1][:, 0]
    ssq1 = m1[0][2][:, 0] + m1[1][2][:, 0]
    mean1 = ssum1 / cnt
    var1 = ssq1 / cnt - mean1 * mean1
    scale1 = g1 / jnp.sqrt(var1 + 1e-5)
    shift1 = be1 - mean1 * scale1
    sc1 = jnp.broadcast_to(scale1[:, None], (D1, 128))
    sh1 = jnp.broadcast_to(shift1[:, None], (D1, 128))

    m2 = [_mm2_call(m1[h][0], W2bf, sc1, sh1, T) for h in range(2)]

    ssum2 = m2[0][1][:, 0] + m2[1][1][:, 0]
    ssq2 = m2[0][2][:, 0] + m2[1][2][:, 0]
    mean2 = ssum2 / cnt
    var2 = ssq2 / cnt - mean2 * mean2
    scale2 = g2 / jnp.sqrt(var2 + 1e-5)
    shift2 = be2 - mean2 * scale2
    sc2 = jnp.broadcast_to(scale2[:, None], (D2, 128))
    sh2 = jnp.broadcast_to(shift2[:, None], (D2, 128))

    return _out_call(m2[0][0], m2[1][0], sc2, sh2, T)


# speedup vs baseline: 1.1034x; 1.1034x over previous
import functools

import jax
import jax.numpy as jnp
from jax import lax
from jax.experimental import pallas as pl
from jax.experimental.pallas import tpu as pltpu
from jax.experimental.pallas import tpu_sc as plsc

_T = 512
_SC_NC = 2
_SC_NS = 16
_SC_CH = 64


def _knn_kernel(n2, p2_ref, p1_ref, q2_ref,
                i0_ref, i1_ref, i2_ref, w0_ref, w1_ref, w2_ref):
    b = pl.program_id(0)
    t = p1_ref.shape[2]

    p1 = p1_ref[0]
    q1 = jnp.sum(p1 * p1, axis=0, keepdims=True)
    cc = lax.dot_general(p2_ref[0].astype(jnp.bfloat16),
                         p1.astype(jnp.bfloat16),
                         (((0,), (0,)), ((), ())),
                         preferred_element_type=jnp.float32)
    d = jnp.maximum((q1 + q2_ref[0]) - 2.0 * cc, 0.0)

    sub = lax.broadcasted_iota(jnp.int32, (n2, t), 0)
    inf = jnp.float32(jnp.inf)
    dcur = d
    ams, ws = [], []
    wsum = jnp.zeros((1, t), jnp.float32)
    for k in range(3):
        m = jnp.min(dcur, axis=0, keepdims=True)
        am = jnp.min(jnp.where(dcur == m, sub, n2), axis=0,
                     keepdims=True)
        wk = 1.0 / (m + 1e-5)
        wsum = wsum + wk
        ams.append(am)
        ws.append(wk)
        if k < 2:
            dcur = jnp.where(sub == am, inf, dcur)

    off = b * n2
    i0_ref[0] = ams[0] + off
    i1_ref[0] = ams[1] + off
    i2_ref[0] = ams[2] + off
    w0_ref[0] = ws[0] / wsum
    w1_ref[0] = ws[1] / wsum
    w2_ref[0] = ws[2] / wsum


def _sc_gather_kernel(nch, table_ref, i0_ref, i1_ref, i2_ref,
                      w0_ref, w1_ref, w2_ref, out_ref,
                      i0a, i1a, i2a, r0a, r1a, r2a, sema,
                      i0b, i1b, i2b, r0b, r1b, r2b, semb,
                      w0v, w1v, w2v, ov):
    ch = _SC_CH
    wid = lax.axis_index("s") * _SC_NC + lax.axis_index("c")
    base = wid * (nch * ch)
    bufs = ((i0a, i1a, i2a, r0a, r1a, r2a, sema),
            (i0b, i1b, i2b, r0b, r1b, r2b, semb))

    def fire(c, slot):
        i0v, i1v, i2v, r0v, r1v, r2v, sem = slot
        off = base + c * ch
        pltpu.sync_copy(i0_ref.at[pl.ds(off, ch)], i0v)
        pltpu.sync_copy(i1_ref.at[pl.ds(off, ch)], i1v)
        pltpu.sync_copy(i2_ref.at[pl.ds(off, ch)], i2v)
        return (pltpu.async_copy(table_ref.at[i0v], r0v, sem),
                pltpu.async_copy(table_ref.at[i1v], r1v, sem),
                pltpu.async_copy(table_ref.at[i2v], r2v, sem))

    def compute(c, slot):
        _, _, _, r0v, r1v, r2v, _ = slot
        off = base + c * ch
        pltpu.sync_copy(w0_ref.at[pl.ds(off, ch)], w0v)
        pltpu.sync_copy(w1_ref.at[pl.ds(off, ch)], w1v)
        pltpu.sync_copy(w2_ref.at[pl.ds(off, ch)], w2v)

        def group_body(jg, carry2):
            w0c = w0v[pl.ds(jg * 16, 16)]
            w1c = w1v[pl.ds(jg * 16, 16)]
            w2c = w2v[pl.ds(jg * 16, 16)]

            def out_body(j16, carry3):
                jl = lax.broadcast(j16, (16,))
                w0s = w0c.at[jl].get(mode="promise_in_bounds")
                w1s = w1c.at[jl].get(mode="promise_in_bounds")
                w2s = w2c.at[jl].get(mode="promise_in_bounds")
                j = jg * 16 + j16
                for cc16 in range(16):
                    sl = pl.ds(cc16 * 16, 16)
                    ov[j, sl] = (r0v[j, sl] * w0s + r1v[j, sl] * w1s
                                 + r2v[j, sl] * w2s)
                return carry3

            lax.fori_loop(0, 16, out_body, 0)
            return carry2

        lax.fori_loop(0, ch // 16, group_body, 0)
        pltpu.sync_copy(ov, out_ref.at[pl.ds(off, ch)])

    handles = {0: fire(0, bufs[0])}
    for c in range(nch):
        if c + 1 < nch:
            handles[(c + 1) % 2] = fire(c + 1, bufs[(c + 1) % 2])
        for h in handles[c % 2]:
            h.wait()
        compute(c, bufs[c % 2])


def _mm1_kernel(nb, ni, f1_ref, g_ref, w1a_ref, w1b_ref,
                y1_ref, ssum_ref, ssq_ref, acc, accsq):
    b = pl.program_id(0)
    i = pl.program_id(1)

    @pl.when((b == 0) & (i == 0))
    def _init():
        acc[...] = jnp.zeros_like(acc)
        accsq[...] = jnp.zeros_like(accsq)

    g = g_ref[0].astype(jnp.bfloat16)
    y = (jnp.dot(w1a_ref[...], f1_ref[0].astype(jnp.bfloat16),
                 preferred_element_type=jnp.float32)
         + lax.dot_general(w1b_ref[...], g, (((1,), (1,)), ((), ())),
                           preferred_element_type=jnp.float32))
    y1_ref[0] = y.astype(jnp.bfloat16)
    acc[...] += y
    accsq[...] += y * y

    @pl.when((b == nb - 1) & (i == ni - 1))
    def _fin():
        ssum_ref[...] = jnp.broadcast_to(
            jnp.sum(acc[...], axis=1, keepdims=True), ssum_ref.shape)
        ssq_ref[...] = jnp.broadcast_to(
            jnp.sum(accsq[...], axis=1, keepdims=True), ssq_ref.shape)


def _bn_mm2_kernel(nb, ni, y1_ref, w2_ref, sc_ref, sh_ref,
                   y2_ref, ssum_ref, ssq_ref, acc, accsq):
    b = pl.program_id(0)
    i = pl.program_id(1)

    @pl.when((b == 0) & (i == 0))
    def _init():
        acc[...] = jnp.zeros_like(acc)
        accsq[...] = jnp.zeros_like(accsq)

    z = jnp.maximum(y1_ref[0].astype(jnp.float32) * sc_ref[:, 0:1]
                    + sh_ref[:, 0:1], 0.0)
    y = jnp.dot(w2_ref[...], z.astype(jnp.bfloat16),
                preferred_element_type=jnp.float32)
    y2_ref[0] = y.astype(jnp.bfloat16)
    acc[...] += y
    accsq[...] += y * y

    @pl.when((b == nb - 1) & (i == ni - 1))
    def _fin():
        ssum_ref[...] = jnp.broadcast_to(
            jnp.sum(acc[...], axis=1, keepdims=True), ssum_ref.shape)
        ssq_ref[...] = jnp.broadcast_to(
            jnp.sum(accsq[...], axis=1, keepdims=True), ssq_ref.shape)


def _bn_out_kernel(y2_ref, sc_ref, sh_ref, o_ref):
    o_ref[0] = jnp.maximum(
        y2_ref[0].astype(jnp.float32) * sc_ref[:, 0:1] + sh_ref[:, 0:1], 0.0)


def _bn_out_kernel2(y2_ref, sc_ref, sh_ref, prev_ref, o_ref):
    o_ref[0] = jnp.maximum(
        y2_ref[0].astype(jnp.float32) * sc_ref[:, 0:1] + sh_ref[:, 0:1], 0.0)


def _knn_call(p2, p1, q2col, N2, T):
    Bh, _, N1 = p1.shape
    i32 = jnp.int32
    f32 = jnp.float32
    grid = (Bh, N1 // T)
    return pl.pallas_call(
        functools.partial(_knn_kernel, N2),
        grid=grid,
        in_specs=[
            pl.BlockSpec((1, 3, N2), lambda b, i: (b, 0, 0)),
            pl.BlockSpec((1, 3, T), lambda b, i: (b, 0, i)),
            pl.BlockSpec((1, N2, 1), lambda b, i: (b, 0, 0)),
        ],
        out_specs=[pl.BlockSpec((1, 1, T), lambda b, i: (b, 0, i))] * 6,
        out_shape=[jax.ShapeDtypeStruct((Bh, 1, N1), i32)] * 3
        + [jax.ShapeDtypeStruct((Bh, 1, N1), f32)] * 3,
    )(p2, p1, q2col)


def _gather_call(table, knn6, C2):
    Bh, _, N1 = knn6[0].shape
    rows = Bh * N1
    nw = _SC_NC * _SC_NS
    nch = rows // (nw * _SC_CH)
    i32 = jnp.int32
    f32 = jnp.float32
    i0, i1, i2, w0, w1, w2 = [a.reshape(rows) for a in knn6]
    gath = pl.kernel(
        functools.partial(_sc_gather_kernel, nch),
        out_type=jax.ShapeDtypeStruct((rows, C2), f32),
        mesh=plsc.VectorSubcoreMesh(core_axis_name="c", subcore_axis_name="s",
                                    num_cores=_SC_NC, num_subcores=_SC_NS),
        scratch_types=(
            [pltpu.VMEM((_SC_CH,), i32)] * 3
            + [pltpu.VMEM((_SC_CH, C2), f32)] * 3
            + [pltpu.SemaphoreType.DMA]
            + [pltpu.VMEM((_SC_CH,), i32)] * 3
            + [pltpu.VMEM((_SC_CH, C2), f32)] * 3
            + [pltpu.SemaphoreType.DMA]
            + [pltpu.VMEM((_SC_CH,), f32)] * 3
            + [pltpu.VMEM((_SC_CH, C2), f32)]
        ),
    )(table, i0, i1, i2, w0, w1, w2)
    return gath.reshape(Bh, N1, C2)


def _mm1_call(f1, f2i, W1a, W1b, T):
    Bh, C1, N1 = f1.shape
    C2 = f2i.shape[2]
    D1 = W1a.shape[0]
    NI = N1 // T
    f32 = jnp.float32
    grid = (Bh, NI)
    return pl.pallas_call(
        functools.partial(_mm1_kernel, Bh, NI),
        grid=grid,
        in_specs=[
            pl.BlockSpec((1, C1, T), lambda b, i: (b, 0, i)),
            pl.BlockSpec((1, T, C2), lambda b, i: (b, i, 0)),
            pl.BlockSpec((D1, C1), lambda b, i: (0, 0)),
            pl.BlockSpec((D1, C2), lambda b, i: (0, 0)),
        ],
        out_specs=[
            pl.BlockSpec((1, D1, T), lambda b, i: (b, 0, i)),
            pl.BlockSpec((D1, 128), lambda b, i: (0, 0)),
            pl.BlockSpec((D1, 128), lambda b, i: (0, 0)),
        ],
        out_shape=[
            jax.ShapeDtypeStruct((Bh, D1, N1), jnp.bfloat16),
            jax.ShapeDtypeStruct((D1, 128), f32),
            jax.ShapeDtypeStruct((D1, 128), f32),
        ],
        scratch_shapes=[
            pltpu.VMEM((D1, T), f32),
            pltpu.VMEM((D1, T), f32),
        ],
    )(f1, f2i, W1a, W1b)


def _mm2_call(y1, W2bf, sc1, sh1, T):
    Bh, D1, N1 = y1.shape
    D2 = W2bf.shape[0]
    NI = N1 // T
    f32 = jnp.float32
    grid = (Bh, NI)
    return pl.pallas_call(
        functools.partial(_bn_mm2_kernel, Bh, NI),
        grid=grid,
        in_specs=[
            pl.BlockSpec((1, D1, T), lambda b, i: (b, 0, i)),
            pl.BlockSpec((D2, D1), lambda b, i: (0, 0)),
            pl.BlockSpec((D1, 128), lambda b, i: (0, 0)),
            pl.BlockSpec((D1, 128), lambda b, i: (0, 0)),
        ],
        out_specs=[
            pl.BlockSpec((1, D2, T), lambda b, i: (b, 0, i)),
            pl.BlockSpec((D2, 128), lambda b, i: (0, 0)),
            pl.BlockSpec((D2, 128), lambda b, i: (0, 0)),
        ],
        out_shape=[
            jax.ShapeDtypeStruct((Bh, D2, N1), jnp.bfloat16),
            jax.ShapeDtypeStruct((D2, 128), f32),
            jax.ShapeDtypeStruct((D2, 128), f32),
        ],
        scratch_shapes=[
            pltpu.VMEM((D2, T), f32),
            pltpu.VMEM((D2, T), f32),
        ],
    )(y1, W2bf, sc1, sh1)


def _out_call(y2a, y2b, sc2, sh2, T):
    H, D2, N1 = y2a.shape
    B = 2 * H
    T = N1 // 2
    grid = (H, N1 // T)
    half = pl.pallas_call(
        _bn_out_kernel,
        grid=grid,
        in_specs=[
            pl.BlockSpec((1, D2, T), lambda b, i: (b, 0, i)),
            pl.BlockSpec((D2, 128), lambda b, i: (0, 0)),
            pl.BlockSpec((D2, 128), lambda b, i: (0, 0)),
        ],
        out_specs=pl.BlockSpec((1, D2, T), lambda b, i: (b, 0, i)),
        out_shape=jax.ShapeDtypeStruct((B, D2, N1), jnp.float32),
    )(y2a, sc2, sh2)
    return pl.pallas_call(
        _bn_out_kernel2,
        grid=grid,
        in_specs=[
            pl.BlockSpec((1, D2, T), lambda b, i: (b, 0, i)),
            pl.BlockSpec((D2, 128), lambda b, i: (0, 0)),
            pl.BlockSpec((D2, 128), lambda b, i: (0, 0)),
            pl.BlockSpec(memory_space=pltpu.MemorySpace.HBM),
        ],
        out_specs=pl.BlockSpec((1, D2, T), lambda b, i: (b + H, 0, i)),
        out_shape=jax.ShapeDtypeStruct((B, D2, N1), jnp.float32),
        input_output_aliases={3: 0},
    )(y2b, sc2, sh2, half)


def kernel(points1, features1, points2, features2,
           W1, b1, g1, be1, W2, b2, g2, be2):
    B, _, N1 = points1.shape
    N2 = points2.shape[2]
    C1 = features1.shape[1]
    C2 = features2.shape[1]
    D1 = W1.shape[0]
    D2 = W2.shape[0]
    T = _T
    f32 = jnp.float32
    bf16 = jnp.bfloat16

    sq2 = jnp.sum(points2 * points2, axis=1, keepdims=True)
    q2col = jnp.transpose(sq2, (0, 2, 1))

    W1a = W1[:, :C1].astype(bf16)
    W1b = W1[:, C1:].astype(bf16)
    W2bf = W2.astype(bf16)

    H = B // 2
    tables = [jnp.transpose(features2[h * H:(h + 1) * H], (0, 2, 1))
              .reshape(H * N2, C2) for h in range(2)]

    knn = [_knn_call(points2[h * H:(h + 1) * H], points1[h * H:(h + 1) * H],
                     q2col[h * H:(h + 1) * H], N2, T) for h in range(2)]
    f2i = [_gather_call(tables[h], knn[h], C2) for h in range(2)]
    m1 = [_mm1_call(features1[h * H:(h + 1) * H], f2i[h], W1a, W1b, T)
          for h in range(2)]

    cnt = f32(B * N1)
    ssum1 = m1[0][1][:, 0] + m1[1][1][:, 0]
    ssq1 = m1[0][2][:, 0] + m1[1][2][:, 0]
    mean1 = ssum1 / cnt
    var1 = ssq1 / cnt - mean1 * mean1
    scale1 = g1 / jnp.sqrt(var1 + 1e-5)
    shift1 = be1 - mean1 * scale1
    sc1 = jnp.broadcast_to(scale1[:, None], (D1, 128))
    sh1 = jnp.broadcast_to(shift1[:, None], (D1, 128))

    m2 = [_mm2_call(m1[h][0], W2bf, sc1, sh1, T) for h in range(2)]

    ssum2 = m2[0][1][:, 0] + m2[1][1][:, 0]
    ssq2 = m2[0][2][:, 0] + m2[1][2][:, 0]
    mean2 = ssum2 / cnt
    var2 = ssq2 / cnt - mean2 * mean2
    scale2 = g2 / jnp.sqrt(var2 + 1e-5)
    shift2 = be2 - mean2 * scale2
    sc2 = jnp.broadcast_to(scale2[:, None], (D2, 128))
    sh2 = jnp.broadcast_to(shift2[:, None], (D2, 128))

    return _out_call(m2[0][0], m2[1][0], sc2, sh2, T)
